# initial kernel scaffold (unmeasured)
import jax
import jax.numpy as jnp
from jax import lax
from jax.experimental import pallas as pl
from jax.experimental.pallas import tpu as pltpu

N_DEV = 16
M_BLK = 256


def kernel(x, w_mat):
    k_total, m_shard = x.shape
    n = w_mat.shape[1]

    def body(x_ref, w_ref, out_ref, comm_ref, amax_ref,
             send_sems, recv_sems, amax_send_sems, amax_recv_sems):
        my = lax.axis_index("i")

        barrier_sem = pltpu.get_barrier_semaphore()
        for d in range(1, N_DEV):
            pl.semaphore_signal(
                barrier_sem, inc=1,
                device_id=((my + d) % N_DEV,),
                device_id_type=pl.DeviceIdType.MESH,
            )
        pl.semaphore_wait(barrier_sem, N_DEV - 1)

        sends = []
        for d in range(1, N_DEV):
            dst = (my + d) % N_DEV
            rdma = pltpu.make_async_remote_copy(
                src_ref=x_ref.at[pl.ds(dst * M_BLK, M_BLK), :],
                dst_ref=comm_ref.at[d],
                send_sem=send_sems.at[d],
                recv_sem=recv_sems.at[d],
                device_id=(dst,),
                device_id_type=pl.DeviceIdType.MESH,
            )
            rdma.start()
            sends.append(rdma)

        out_ref[:, :] = jnp.dot(
            x_ref[pl.ds(my * M_BLK, M_BLK), :],
            w_ref[pl.ds(my * M_BLK, M_BLK), :],
            preferred_element_type=jnp.float32,
        )

        for d in range(1, N_DEV):
            src = (my - d) % N_DEV
            sends[d - 1].wait_recv()
            out_ref[:, :] += jnp.dot(
                comm_ref[d],
                w_ref[pl.ds(src * M_BLK, M_BLK), :],
                preferred_element_type=jnp.float32,
            )

        for rdma in sends:
            rdma.wait_send()

        y = jnp.maximum(out_ref[:, :], 0.0)
        out_ref[:, :] = y
        amax_ref[0, :, :] = jnp.full((8, 128), jnp.max(y), dtype=jnp.float32)

        amax_sends = []
        for d in range(1, N_DEV):
            dst = (my + d) % N_DEV
            rdma = pltpu.make_async_remote_copy(
                src_ref=amax_ref.at[0],
                dst_ref=amax_ref.at[d],
                send_sem=amax_send_sems.at[d],
                recv_sem=amax_recv_sems.at[d],
                device_id=(dst,),
                device_id_type=pl.DeviceIdType.MESH,
            )
            rdma.start()
            amax_sends.append(rdma)
        for rdma in amax_sends:
            rdma.wait_recv()
        for rdma in amax_sends:
            rdma.wait_send()

        scale = jnp.max(amax_ref[:, :, :]) / 127.0
        q = jnp.clip(jnp.round(out_ref[:, :] / scale), -127.0, 127.0)
        out_ref[:, :] = q * scale

    return pl.pallas_call(
        body,
        out_shape=jax.ShapeDtypeStruct((M_BLK, n), jnp.float32),
        in_specs=[
            pl.BlockSpec(memory_space=pltpu.VMEM),
            pl.BlockSpec(memory_space=pltpu.VMEM),
        ],
        out_specs=pl.BlockSpec(memory_space=pltpu.VMEM),
        scratch_shapes=[
            pltpu.VMEM((N_DEV, M_BLK, m_shard), jnp.float32),
            pltpu.VMEM((N_DEV, 8, 128), jnp.float32),
            pltpu.SemaphoreType.DMA((N_DEV,)),
            pltpu.SemaphoreType.DMA((N_DEV,)),
            pltpu.SemaphoreType.DMA((N_DEV,)),
            pltpu.SemaphoreType.DMA((N_DEV,)),
        ],
        compiler_params=pltpu.CompilerParams(collective_id=0),
    )(x, w_mat)


# baseline (device time: 81236 ns/iter reference)
import jax
import jax.numpy as jnp
from jax import lax
from jax.experimental import pallas as pl
from jax.experimental.pallas import tpu as pltpu

N_DEV = 16
M_BLK = 256


def kernel(x, w_mat):
    k_total, m_shard = x.shape
    n = w_mat.shape[1]

    def body(x_ref, w_ref, out_ref, comm_ref, amax_ref,
             send_sems, recv_sems, amax_send_sems, amax_recv_sems):
        my = lax.axis_index("i")

        barrier_sem = pltpu.get_barrier_semaphore()
        for d in range(1, N_DEV):
            pl.semaphore_signal(
                barrier_sem, inc=1,
                device_id=((my + d) % N_DEV,),
                device_id_type=pl.DeviceIdType.MESH,
            )
        pl.semaphore_wait(barrier_sem, N_DEV - 1)

        sends = []
        for d in range(1, N_DEV):
            dst = (my + d) % N_DEV
            rdma = pltpu.make_async_remote_copy(
                src_ref=x_ref.at[pl.ds(dst * M_BLK, M_BLK), :],
                dst_ref=comm_ref.at[d],
                send_sem=send_sems.at[d],
                recv_sem=recv_sems.at[d],
                device_id=(dst,),
                device_id_type=pl.DeviceIdType.MESH,
            )
            rdma.start()
            sends.append(rdma)

        out_ref[:, :] = jnp.dot(
            x_ref[pl.ds(my * M_BLK, M_BLK), :],
            w_ref[pl.ds(my * M_BLK, M_BLK), :],
            preferred_element_type=jnp.float32,
        )

        for d in range(1, N_DEV):
            src = (my - d) % N_DEV
            sends[d - 1].wait_recv()
            out_ref[:, :] += jnp.dot(
                comm_ref[d],
                w_ref[pl.ds(src * M_BLK, M_BLK), :],
                preferred_element_type=jnp.float32,
            )

        for rdma in sends:
            rdma.wait_send()

        y = jnp.maximum(out_ref[:, :], 0.0)
        out_ref[:, :] = y
        amax_ref[0, :, :] = jnp.full((8, 128), jnp.max(y), dtype=jnp.float32)

        amax_sends = []
        for d in range(1, N_DEV):
            dst = (my + d) % N_DEV
            rdma = pltpu.make_async_remote_copy(
                src_ref=amax_ref.at[0],
                dst_ref=amax_ref.at[d],
                send_sem=amax_send_sems.at[d],
                recv_sem=amax_recv_sems.at[d],
                device_id=(dst,),
                device_id_type=pl.DeviceIdType.MESH,
            )
            rdma.start()
            amax_sends.append(rdma)
        for rdma in amax_sends:
            rdma.wait_recv()
        for rdma in amax_sends:
            rdma.wait_send()

        scale = jnp.max(amax_ref[:, :, :]) / 127.0
        q = jnp.clip(jnp.round(out_ref[:, :] / scale), -127.0, 127.0)
        out_ref[:, :] = q * scale

    return pl.pallas_call(
        body,
        out_shape=jax.ShapeDtypeStruct((M_BLK, n), jnp.float32),
        in_specs=[
            pl.BlockSpec(memory_space=pltpu.VMEM),
            pl.BlockSpec(memory_space=pltpu.VMEM),
        ],
        out_specs=pl.BlockSpec(memory_space=pltpu.VMEM),
        scratch_shapes=[
            pltpu.VMEM((N_DEV, M_BLK, m_shard), jnp.float32),
            pltpu.VMEM((N_DEV, 8, 128), jnp.float32),
            pltpu.SemaphoreType.DMA((N_DEV,)),
            pltpu.SemaphoreType.DMA((N_DEV,)),
            pltpu.SemaphoreType.DMA((N_DEV,)),
            pltpu.SemaphoreType.DMA((N_DEV,)),
        ],
        compiler_params=pltpu.CompilerParams(
            collective_id=0,
            vmem_limit_bytes=100 * 1024 * 1024,
        ),
    )(x, w_mat)
